# baseline (speedup 1.0000x reference)
import functools

import jax
import jax.numpy as jnp
from jax import lax
from jax.experimental import pallas as pl
from jax.experimental.pallas import tpu as pltpu

D_MODEL = 2048
N_EXP = 16
BLK_T = 1024


def _router_kernel(x_ref, wt_ref, b_ref, idx_ref, score_ref, aux_ref,
                   cnt_ref, ps_ref, *, num_blocks, num_tokens):
    i = pl.program_id(0)

    @pl.when(i == 0)
    def _init():
        cnt_ref[...] = jnp.zeros_like(cnt_ref)
        ps_ref[...] = jnp.zeros_like(ps_ref)

    logits = jnp.dot(x_ref[...], wt_ref[...],
                     preferred_element_type=jnp.float32) + b_ref[...]
    m = jnp.max(logits, axis=-1, keepdims=True)
    e = jnp.exp(logits - m)
    probs = e / jnp.sum(e, axis=-1, keepdims=True)

    eidx = lax.broadcasted_iota(jnp.int32, probs.shape, 1)
    max1 = jnp.max(probs, axis=-1, keepdims=True)
    idx1 = jnp.min(jnp.where(probs == max1, eidx, N_EXP),
                   axis=-1, keepdims=True)
    hit1 = eidx == idx1
    masked = jnp.where(hit1, -1.0, probs)
    max2 = jnp.max(masked, axis=-1, keepdims=True)
    idx2 = jnp.min(jnp.where(masked == max2, eidx, N_EXP),
                   axis=-1, keepdims=True)
    hit2 = eidx == idx2

    idx_ref[...] = jnp.concatenate([idx1, idx2], axis=-1)
    score_ref[...] = jnp.concatenate([max1, max2], axis=-1)

    cnt_ref[...] += jnp.sum(hit1.astype(jnp.float32) + hit2.astype(jnp.float32),
                            axis=0, keepdims=True)
    ps_ref[...] += jnp.sum(probs, axis=0, keepdims=True)

    @pl.when(i == num_blocks - 1)
    def _fin():
        inv = 1.0 / num_tokens
        aux_ref[...] = N_EXP * jnp.sum(
            (cnt_ref[...] * inv) * (ps_ref[...] * inv), keepdims=True)


@functools.partial(jax.jit, static_argnames=())
def kernel(x, W, b):
    B, S, D = x.shape
    num_tokens = B * S
    num_blocks = num_tokens // BLK_T
    xf = x.reshape(num_tokens, D)
    wt = W.T
    b2 = b.reshape(1, N_EXP)

    idx, score, aux = pl.pallas_call(
        functools.partial(_router_kernel, num_blocks=num_blocks,
                          num_tokens=num_tokens),
        grid=(num_blocks,),
        in_specs=[
            pl.BlockSpec((BLK_T, D), lambda i: (i, 0)),
            pl.BlockSpec((D, N_EXP), lambda i: (0, 0)),
            pl.BlockSpec((1, N_EXP), lambda i: (0, 0)),
        ],
        out_specs=[
            pl.BlockSpec((BLK_T, 2), lambda i: (i, 0)),
            pl.BlockSpec((BLK_T, 2), lambda i: (i, 0)),
            pl.BlockSpec((1, 1), lambda i: (0, 0)),
        ],
        out_shape=[
            jax.ShapeDtypeStruct((num_tokens, 2), jnp.int32),
            jax.ShapeDtypeStruct((num_tokens, 2), jnp.float32),
            jax.ShapeDtypeStruct((1, 1), jnp.float32),
        ],
        scratch_shapes=[
            pltpu.VMEM((1, N_EXP), jnp.float32),
            pltpu.VMEM((1, N_EXP), jnp.float32),
        ],
        compiler_params=pltpu.CompilerParams(
            dimension_semantics=("arbitrary",),
        ),
    )(xf, wt, b2)

    return (idx.reshape(B, S, 2), score.reshape(B, S, 2), aux[0, 0])


# speedup vs baseline: 1.0458x; 1.0458x over previous
import functools

import jax
import jax.numpy as jnp
from jax import lax
from jax.experimental import pallas as pl
from jax.experimental.pallas import tpu as pltpu

D_MODEL = 2048
N_EXP = 16
BLK_T = 2048


def _router_kernel(x_ref, wt_ref, b_ref, idx_ref, score_ref, aux_ref,
                   cnt_ref, ps_ref, *, num_blocks, num_tokens):
    i = pl.program_id(0)

    @pl.when(i == 0)
    def _init():
        cnt_ref[...] = jnp.zeros_like(cnt_ref)
        ps_ref[...] = jnp.zeros_like(ps_ref)

    logits = jnp.dot(x_ref[...], wt_ref[...],
                     preferred_element_type=jnp.float32) + b_ref[...]
    m = jnp.max(logits, axis=-1, keepdims=True)
    e = jnp.exp(logits - m)
    probs = e / jnp.sum(e, axis=-1, keepdims=True)

    eidx = lax.broadcasted_iota(jnp.int32, probs.shape, 1)
    max1 = jnp.max(probs, axis=-1, keepdims=True)
    idx1 = jnp.min(jnp.where(probs == max1, eidx, N_EXP),
                   axis=-1, keepdims=True)
    hit1 = eidx == idx1
    masked = jnp.where(hit1, -1.0, probs)
    max2 = jnp.max(masked, axis=-1, keepdims=True)
    idx2 = jnp.min(jnp.where(masked == max2, eidx, N_EXP),
                   axis=-1, keepdims=True)
    hit2 = eidx == idx2

    idx_ref[...] = jnp.concatenate([idx1, idx2], axis=-1)
    score_ref[...] = jnp.concatenate([max1, max2], axis=-1)

    cnt_ref[...] += jnp.sum(hit1.astype(jnp.float32) + hit2.astype(jnp.float32),
                            axis=0, keepdims=True)
    ps_ref[...] += jnp.sum(probs, axis=0, keepdims=True)

    @pl.when(i == num_blocks - 1)
    def _fin():
        inv = 1.0 / num_tokens
        aux_ref[...] = N_EXP * jnp.sum(
            (cnt_ref[...] * inv) * (ps_ref[...] * inv), keepdims=True)


@functools.partial(jax.jit, static_argnames=())
def kernel(x, W, b):
    B, S, D = x.shape
    num_tokens = B * S
    num_blocks = num_tokens // BLK_T
    xf = x.reshape(num_tokens, D)
    wt = W.T
    b2 = b.reshape(1, N_EXP)

    idx, score, aux = pl.pallas_call(
        functools.partial(_router_kernel, num_blocks=num_blocks,
                          num_tokens=num_tokens),
        grid=(num_blocks,),
        in_specs=[
            pl.BlockSpec((BLK_T, D), lambda i: (i, 0)),
            pl.BlockSpec((D, N_EXP), lambda i: (0, 0)),
            pl.BlockSpec((1, N_EXP), lambda i: (0, 0)),
        ],
        out_specs=[
            pl.BlockSpec((BLK_T, 2), lambda i: (i, 0)),
            pl.BlockSpec((BLK_T, 2), lambda i: (i, 0)),
            pl.BlockSpec((1, 1), lambda i: (0, 0)),
        ],
        out_shape=[
            jax.ShapeDtypeStruct((num_tokens, 2), jnp.int32),
            jax.ShapeDtypeStruct((num_tokens, 2), jnp.float32),
            jax.ShapeDtypeStruct((1, 1), jnp.float32),
        ],
        scratch_shapes=[
            pltpu.VMEM((1, N_EXP), jnp.float32),
            pltpu.VMEM((1, N_EXP), jnp.float32),
        ],
        compiler_params=pltpu.CompilerParams(
            dimension_semantics=("arbitrary",),
        ),
    )(xf, wt, b2)

    return (idx.reshape(B, S, 2), score.reshape(B, S, 2), aux[0, 0])
